# baseline (speedup 1.0000x reference)
import functools

import jax
import jax.numpy as jnp
from jax import lax
from jax.experimental import pallas as pl
from jax.experimental.pallas import tpu as pltpu
from jax.experimental.pallas import tpu_sc as plsc

N = 10000
E = 320000
D = 128
NP = 10240
HALF = NP // 2
NC = 2
NS = 16
NW = NC * NS
L = 16
CH = 64
T = 160
EPW = T * CH
EP = NW * EPW
ROWS_PER_TILE = NP // NS

_mesh = plsc.VectorSubcoreMesh(core_axis_name="c", subcore_axis_name="s")
_HI = lax.Precision.HIGHEST


@functools.partial(
    pl.kernel,
    mesh=_mesh,
    out_type=jax.ShapeDtypeStruct((NW, NP * L), jnp.float32),
    scratch_types=[
        pltpu.VMEM((EPW,), jnp.int32),
        pltpu.VMEM((EPW,), jnp.float32),
        pltpu.VMEM((HALF * L,), jnp.float32),
    ],
    compiler_params=pltpu.CompilerParams(needs_layout_passes=False),
)
def _deg_sc(dst_hbm, ew_hbm, out_hbm, dst_v, ew_v, degw):
    c = lax.axis_index("c")
    s = lax.axis_index("s")
    w = c * NS + s
    pltpu.sync_copy(dst_hbm.at[w], dst_v)
    pltpu.sync_copy(ew_hbm.at[w], ew_v)
    col = lax.iota(jnp.int32, L)
    for half in range(2):
        lo = half * HALF

        def zbody(i, carry):
            for u in range(8):
                degw[pl.ds((i * 8 + u) * L, L)] = jnp.zeros((L,), jnp.float32)
            return carry

        lax.fori_loop(0, HALF // 8, zbody, 0)

        def ebody(g, carry):
            for u in range(4):
                d = dst_v[pl.ds((g * 4 + u) * L, L)]
                wv = ew_v[pl.ds((g * 4 + u) * L, L)]
                idx = (d - lo) * L + col
                m = (d >= lo) & (d < lo + HALF)
                plsc.addupdate_scatter(degw, [idx], wv, mask=m)
            return carry

        lax.fori_loop(0, EPW // L // 4, ebody, 0)
        pltpu.sync_copy(degw, out_hbm.at[w, pl.ds(lo * L, HALF * L)])


@functools.partial(
    pl.kernel,
    mesh=_mesh,
    out_type=jax.ShapeDtypeStruct((NC, NP, D), jnp.float32),
    scratch_types=[
        pltpu.VMEM((T, CH), jnp.int32),
        pltpu.VMEM((T, CH), jnp.int32),
        pltpu.VMEM((EPW,), jnp.float32),
        pltpu.VMEM((CH, D), jnp.float32),
        pltpu.VMEM((CH, D), jnp.float32),
        pltpu.VMEM_SHARED((NP, D), jnp.float32),
        pltpu.SemaphoreType.DMA,
        pltpu.SemaphoreType.DMA,
    ],
    compiler_params=pltpu.CompilerParams(needs_layout_passes=False,
                                         use_tc_tiling_on_sc=False),
)
def _edge_sc(y_hbm, src_hbm, dst_hbm, ew_hbm, out_hbm,
             src_v, dst_v, ew_v, rows0_v, rows1_v, acc_sh, sem0, sem1):
    c = lax.axis_index("c")
    s = lax.axis_index("s")
    w = c * NS + s

    def zbody(i, carry):
        for f in range(D // L):
            rows0_v[i, pl.ds(f * L, L)] = jnp.zeros((L,), jnp.float32)
        return carry

    lax.fori_loop(0, CH, zbody, 0)
    for k in range(ROWS_PER_TILE // CH):
        pltpu.sync_copy(rows0_v, acc_sh.at[pl.ds(s * ROWS_PER_TILE + k * CH, CH)])

    pltpu.sync_copy(src_hbm.at[w], src_v)
    pltpu.sync_copy(dst_hbm.at[w], dst_v)
    pltpu.sync_copy(ew_hbm.at[w], ew_v)
    plsc.subcore_barrier()

    def scale_scatter(j, rows_v):
        def ebody(e, icarry):
            for u in range(2):
                ee = e * 2 + u
                wv = plsc.load_gather(
                    ew_v, [jnp.full((L,), j * CH + ee, jnp.int32)])
                for f in range(D // L):
                    sl = pl.ds(f * L, L)
                    rows_v[ee, sl] = rows_v[ee, sl] * wv
            return icarry

        lax.fori_loop(0, CH // 2, ebody, 0)
        pltpu.sync_copy(rows_v, acc_sh.at[dst_v.at[j]], add=True)

    pltpu.async_copy(y_hbm.at[src_v.at[0]], rows0_v, sem0)

    def pair(p, carry):
        j0 = p * 2
        j1 = j0 + 1
        pltpu.make_async_copy(y_hbm.at[src_v.at[j0]], rows0_v, sem0).wait()
        pltpu.async_copy(y_hbm.at[src_v.at[j1]], rows1_v, sem1)
        scale_scatter(j0, rows0_v)
        pltpu.make_async_copy(y_hbm.at[src_v.at[j1]], rows1_v, sem1).wait()
        jn = lax.rem(j0 + 2, T)
        pltpu.async_copy(y_hbm.at[src_v.at[jn]], rows0_v, sem0)
        scale_scatter(j1, rows1_v)
        return carry

    lax.fori_loop(0, T // 2, pair, 0)
    pltpu.make_async_copy(y_hbm.at[src_v.at[0]], rows0_v, sem0).wait()
    plsc.subcore_barrier()
    pltpu.sync_copy(acc_sh.at[pl.ds(s * ROWS_PER_TILE, ROWS_PER_TILE)],
                    out_hbm.at[c, pl.ds(s * ROWS_PER_TILE, ROWS_PER_TILE)])


def _gru_math(W, wih, whh, bih, bhh):
    gx = lax.dot_general(W, wih, (((1,), (1,)), ((), ())), precision=_HI)
    gx = gx + bih[None, :]
    gh = lax.dot_general(W, whh, (((1,), (1,)), ((), ())), precision=_HI)
    gh = gh + bhh[None, :]
    d = W.shape[1]
    r = jax.nn.sigmoid(gx[:, :d] + gh[:, :d])
    z = jax.nn.sigmoid(gx[:, d:2 * d] + gh[:, d:2 * d])
    n = jnp.tanh(gx[:, 2 * d:] + r * gh[:, 2 * d:])
    return (1.0 - z) * n + z * W


def _gru_body(W0r, wih0, whh0, bih0, bhh0, W1r, wih1, whh1, bih1, bhh1,
              Wa_ref, Wb_ref):
    Wa_ref[...] = _gru_math(W0r[...], wih0[...], whh0[...], bih0[...], bhh0[...])
    Wb_ref[...] = _gru_math(W1r[...], wih1[...], whh1[...], bih1[...], bhh1[...])


def _gru_call(W0, g0wi, g0wh, g0bi, g0bh, W1, g1wi, g1wh, g1bi, g1bh):
    return pl.pallas_call(
        _gru_body,
        out_shape=(jax.ShapeDtypeStruct((D, D), jnp.float32),
                   jax.ShapeDtypeStruct((D, D), jnp.float32)),
    )(W0, g0wi, g0wh, g0bi, g0bh, W1, g1wi, g1wh, g1bi, g1bh)


_BLK = 1024
_G = NP // _BLK


def _prep_body(degp_ref, x_ref, Wa_ref, y0_ref, dinv_ref):
    degp = degp_ref[...].reshape(NW, _BLK, L)
    deg = jnp.sum(degp, axis=(0, 2)) + 1.0
    dinv = lax.rsqrt(deg)
    xw = lax.dot_general(x_ref[...], Wa_ref[...], (((1,), (0,)), ((), ())),
                         precision=_HI)
    y0_ref[...] = xw * dinv[:, None]
    dinv_ref[...] = dinv


def _prep_call(degp, x_p, Wa):
    return pl.pallas_call(
        _prep_body,
        grid=(_G,),
        in_specs=[
            pl.BlockSpec((NW, _BLK * L), lambda i: (0, i)),
            pl.BlockSpec((_BLK, D), lambda i: (i, 0)),
            pl.BlockSpec((D, D), lambda i: (0, 0)),
        ],
        out_specs=[
            pl.BlockSpec((_BLK, D), lambda i: (i, 0)),
            pl.BlockSpec((_BLK,), lambda i: (i,)),
        ],
        out_shape=(jax.ShapeDtypeStruct((NP, D), jnp.float32),
                   jax.ShapeDtypeStruct((NP,), jnp.float32)),
    )(degp, x_p, Wa)


def _mid_body(a_ref, y0_ref, dinv_ref, l0w_ref, l0b_ref, Wb_ref, y1_ref):
    dinv = dinv_ref[...][:, None]
    t = (a_ref[0] + a_ref[1] + y0_ref[...]) * dinv
    h = jnp.maximum(t, 0.0)
    h1 = lax.dot_general(h, l0w_ref[...], (((1,), (1,)), ((), ())),
                         precision=_HI) + l0b_ref[...][None, :]
    y1_ref[...] = lax.dot_general(h1, Wb_ref[...], (((1,), (0,)), ((), ())),
                                  precision=_HI) * dinv


def _mid_call(acc, y0, dinv, l0w, l0b, Wb):
    return pl.pallas_call(
        _mid_body,
        grid=(_G,),
        in_specs=[
            pl.BlockSpec((NC, _BLK, D), lambda i: (0, i, 0)),
            pl.BlockSpec((_BLK, D), lambda i: (i, 0)),
            pl.BlockSpec((_BLK,), lambda i: (i,)),
            pl.BlockSpec((D, D), lambda i: (0, 0)),
            pl.BlockSpec((D,), lambda i: (0,)),
            pl.BlockSpec((D, D), lambda i: (0, 0)),
        ],
        out_specs=pl.BlockSpec((_BLK, D), lambda i: (i, 0)),
        out_shape=jax.ShapeDtypeStruct((NP, D), jnp.float32),
    )(acc, y0, dinv, l0w, l0b, Wb)


def _fin_body(a_ref, y1_ref, dinv_ref, l1w_ref, l1b_ref, o_ref):
    dinv = dinv_ref[...][:, None]
    t = (a_ref[0] + a_ref[1] + y1_ref[...]) * dinv
    o = lax.dot_general(t, l1w_ref[...], (((1,), (1,)), ((), ())),
                        precision=_HI) + l1b_ref[...][None, :]
    o_ref[...] = jax.nn.sigmoid(o)


def _fin_call(acc, y1, dinv, l1w_p, l1b_p):
    return pl.pallas_call(
        _fin_body,
        grid=(_G,),
        in_specs=[
            pl.BlockSpec((NC, _BLK, D), lambda i: (0, i, 0)),
            pl.BlockSpec((_BLK, D), lambda i: (i, 0)),
            pl.BlockSpec((_BLK,), lambda i: (i,)),
            pl.BlockSpec((D, D), lambda i: (0, 0)),
            pl.BlockSpec((D,), lambda i: (0,)),
        ],
        out_specs=pl.BlockSpec((_BLK, D), lambda i: (i, 0)),
        out_shape=jax.ShapeDtypeStruct((NP, D), jnp.float32),
    )(acc, y1, dinv, l1w_p, l1b_p)


def kernel(x, edge_index, edge_weight, W0, gru0_w_ih, gru0_w_hh, gru0_b_ih,
           gru0_b_hh, lin0_w, lin0_b, W1, gru1_w_ih, gru1_w_hh, gru1_b_ih,
           gru1_b_hh, lin1_w, lin1_b):
    src = edge_index[0].astype(jnp.int32)
    dst = edge_index[1].astype(jnp.int32)
    pad = EP - E
    src_p = jnp.concatenate([src, jnp.zeros((pad,), jnp.int32)])
    dst_p = jnp.concatenate([dst, jnp.zeros((pad,), jnp.int32)])
    ew_p = jnp.concatenate([edge_weight, jnp.zeros((pad,), jnp.float32)])
    src3 = src_p.reshape(NW, T, CH)
    dst3 = dst_p.reshape(NW, T, CH)
    dst2 = dst_p.reshape(NW, EPW)
    ew2 = ew_p.reshape(NW, EPW)
    x_p = jnp.concatenate([x, jnp.zeros((NP - N, D), jnp.float32)])
    l1w_p = jnp.zeros((D, D), jnp.float32).at[: lin1_w.shape[0]].set(lin1_w)
    l1b_p = jnp.zeros((D,), jnp.float32).at[: lin1_b.shape[0]].set(lin1_b)

    Wa, Wb = _gru_call(W0, gru0_w_ih, gru0_w_hh, gru0_b_ih, gru0_b_hh,
                       W1, gru1_w_ih, gru1_w_hh, gru1_b_ih, gru1_b_hh)
    degp = _deg_sc(dst2, ew2)
    y0, dinv = _prep_call(degp, x_p, Wa)
    acc0 = _edge_sc(y0, src3, dst3, ew2)
    y1 = _mid_call(acc0, y0, dinv, lin0_w, lin0_b, Wb)
    acc1 = _edge_sc(y1, src3, dst3, ew2)
    o = _fin_call(acc1, y1, dinv, l1w_p, l1b_p)
    return o[:N, : lin1_w.shape[0]]


# speedup vs baseline: 1.2173x; 1.2173x over previous
import functools

import jax
import jax.numpy as jnp
from jax import lax
from jax.experimental import pallas as pl
from jax.experimental.pallas import tpu as pltpu
from jax.experimental.pallas import tpu_sc as plsc

N = 10000
E = 320000
D = 128
NP = 10240
HALF = NP // 2
NC = 2
NS = 16
NW = NC * NS
L = 16
CH = 64
T = 160
EPW = T * CH
EP = NW * EPW
ROWS_PER_TILE = NP // NS

_mesh = plsc.VectorSubcoreMesh(core_axis_name="c", subcore_axis_name="s")
_HI = lax.Precision.HIGHEST


@functools.partial(
    pl.kernel,
    mesh=_mesh,
    out_type=jax.ShapeDtypeStruct((NW, NP * L), jnp.float32),
    scratch_types=[
        pltpu.VMEM((EPW,), jnp.int32),
        pltpu.VMEM((EPW,), jnp.float32),
        pltpu.VMEM((HALF * L,), jnp.float32),
    ],
    compiler_params=pltpu.CompilerParams(needs_layout_passes=False),
)
def _deg_sc(dst_hbm, ew_hbm, out_hbm, dst_v, ew_v, degw):
    c = lax.axis_index("c")
    s = lax.axis_index("s")
    w = c * NS + s
    pltpu.sync_copy(dst_hbm.at[w], dst_v)
    pltpu.sync_copy(ew_hbm.at[w], ew_v)
    col = lax.iota(jnp.int32, L)
    for half in range(2):
        lo = half * HALF

        def zbody(i, carry):
            for u in range(8):
                degw[pl.ds((i * 8 + u) * L, L)] = jnp.zeros((L,), jnp.float32)
            return carry

        lax.fori_loop(0, HALF // 8, zbody, 0)

        def ebody(g, carry):
            for u in range(4):
                d = dst_v[pl.ds((g * 4 + u) * L, L)]
                wv = ew_v[pl.ds((g * 4 + u) * L, L)]
                idx = (d - lo) * L + col
                m = (d >= lo) & (d < lo + HALF)
                plsc.addupdate_scatter(degw, [idx], wv, mask=m)
            return carry

        lax.fori_loop(0, EPW // L // 4, ebody, 0)
        pltpu.sync_copy(degw, out_hbm.at[w, pl.ds(lo * L, HALF * L)])


@functools.partial(
    pl.kernel,
    mesh=_mesh,
    out_type=jax.ShapeDtypeStruct((NC, NP, D), jnp.float32),
    scratch_types=[
        pltpu.VMEM((T, CH), jnp.int32),
        pltpu.VMEM((T, CH), jnp.int32),
        pltpu.VMEM((EPW,), jnp.float32),
        pltpu.VMEM((CH, D), jnp.bfloat16),
        pltpu.VMEM((CH, D), jnp.bfloat16),
        pltpu.VMEM((CH, D), jnp.float32),
        pltpu.VMEM_SHARED((NP, D), jnp.float32),
        pltpu.SemaphoreType.DMA,
        pltpu.SemaphoreType.DMA,
    ],
    compiler_params=pltpu.CompilerParams(needs_layout_passes=False,
                                         use_tc_tiling_on_sc=False),
)
def _edge_sc(y_hbm, src_hbm, dst_hbm, ew_hbm, out_hbm,
             src_v, dst_v, ew_v, rows0_v, rows1_v, rowsf_v, acc_sh,
             sem0, sem1):
    c = lax.axis_index("c")
    s = lax.axis_index("s")
    w = c * NS + s

    def zbody(i, carry):
        for f in range(D // L):
            rowsf_v[i, pl.ds(f * L, L)] = jnp.zeros((L,), jnp.float32)
        return carry

    lax.fori_loop(0, CH, zbody, 0)
    for k in range(ROWS_PER_TILE // CH):
        pltpu.sync_copy(rowsf_v, acc_sh.at[pl.ds(s * ROWS_PER_TILE + k * CH, CH)])

    pltpu.sync_copy(src_hbm.at[w], src_v)
    pltpu.sync_copy(dst_hbm.at[w], dst_v)
    pltpu.sync_copy(ew_hbm.at[w], ew_v)
    plsc.subcore_barrier()

    col2 = lax.iota(jnp.int32, L) * 2

    def scale_scatter(j, rows_v):
        def ebody(e, icarry):
            wv = plsc.load_gather(
                ew_v, [jnp.full((L,), j * CH + e, jnp.int32)])
            erow = jnp.full((L,), e, jnp.int32)
            for f in range(D // 32):
                xb = rows_v[e, pl.ds(f * 32, 32)]
                xi = plsc.bitcast(xb, jnp.int32)
                fe = plsc.bitcast(lax.shift_left(xi, 16), jnp.float32) * wv
                fo = plsc.bitcast(xi & jnp.int32(-65536), jnp.float32) * wv
                plsc.store_scatter(rowsf_v, [erow, col2 + (32 * f)], fe)
                plsc.store_scatter(rowsf_v, [erow, col2 + (32 * f + 1)], fo)
            return icarry

        lax.fori_loop(0, CH, ebody, 0)
        pltpu.sync_copy(rowsf_v, acc_sh.at[dst_v.at[j]], add=True)

    pltpu.async_copy(y_hbm.at[src_v.at[0]], rows0_v, sem0)

    def pair(p, carry):
        j0 = p * 2
        j1 = j0 + 1
        pltpu.make_async_copy(y_hbm.at[src_v.at[j0]], rows0_v, sem0).wait()
        pltpu.async_copy(y_hbm.at[src_v.at[j1]], rows1_v, sem1)
        scale_scatter(j0, rows0_v)
        pltpu.make_async_copy(y_hbm.at[src_v.at[j1]], rows1_v, sem1).wait()
        jn = lax.rem(j0 + 2, T)
        pltpu.async_copy(y_hbm.at[src_v.at[jn]], rows0_v, sem0)
        scale_scatter(j1, rows1_v)
        return carry

    lax.fori_loop(0, T // 2, pair, 0)
    pltpu.make_async_copy(y_hbm.at[src_v.at[0]], rows0_v, sem0).wait()
    plsc.subcore_barrier()
    pltpu.sync_copy(acc_sh.at[pl.ds(s * ROWS_PER_TILE, ROWS_PER_TILE)],
                    out_hbm.at[c, pl.ds(s * ROWS_PER_TILE, ROWS_PER_TILE)])


def _gru_math(W, wih, whh, bih, bhh):
    gx = lax.dot_general(W, wih, (((1,), (1,)), ((), ())), precision=_HI)
    gx = gx + bih[None, :]
    gh = lax.dot_general(W, whh, (((1,), (1,)), ((), ())), precision=_HI)
    gh = gh + bhh[None, :]
    d = W.shape[1]
    r = jax.nn.sigmoid(gx[:, :d] + gh[:, :d])
    z = jax.nn.sigmoid(gx[:, d:2 * d] + gh[:, d:2 * d])
    n = jnp.tanh(gx[:, 2 * d:] + r * gh[:, 2 * d:])
    return (1.0 - z) * n + z * W


def _gru_body(W0r, wih0, whh0, bih0, bhh0, W1r, wih1, whh1, bih1, bhh1,
              Wa_ref, Wb_ref):
    Wa_ref[...] = _gru_math(W0r[...], wih0[...], whh0[...], bih0[...], bhh0[...])
    Wb_ref[...] = _gru_math(W1r[...], wih1[...], whh1[...], bih1[...], bhh1[...])


def _gru_call(W0, g0wi, g0wh, g0bi, g0bh, W1, g1wi, g1wh, g1bi, g1bh):
    return pl.pallas_call(
        _gru_body,
        out_shape=(jax.ShapeDtypeStruct((D, D), jnp.float32),
                   jax.ShapeDtypeStruct((D, D), jnp.float32)),
    )(W0, g0wi, g0wh, g0bi, g0bh, W1, g1wi, g1wh, g1bi, g1bh)


_BLK = 1024
_G = NP // _BLK


def _prep_body(degp_ref, x_ref, Wa_ref, y0_ref, dinv_ref):
    degp = degp_ref[...].reshape(NW, _BLK, L)
    deg = jnp.sum(degp, axis=(0, 2)) + 1.0
    dinv = lax.rsqrt(deg)
    xw = lax.dot_general(x_ref[...], Wa_ref[...], (((1,), (0,)), ((), ())),
                         precision=_HI)
    y0_ref[...] = (xw * dinv[:, None]).astype(jnp.bfloat16)
    dinv_ref[...] = dinv


def _prep_call(degp, x_p, Wa):
    return pl.pallas_call(
        _prep_body,
        grid=(_G,),
        in_specs=[
            pl.BlockSpec((NW, _BLK * L), lambda i: (0, i)),
            pl.BlockSpec((_BLK, D), lambda i: (i, 0)),
            pl.BlockSpec((D, D), lambda i: (0, 0)),
        ],
        out_specs=[
            pl.BlockSpec((_BLK, D), lambda i: (i, 0)),
            pl.BlockSpec((_BLK,), lambda i: (i,)),
        ],
        out_shape=(jax.ShapeDtypeStruct((NP, D), jnp.bfloat16),
                   jax.ShapeDtypeStruct((NP,), jnp.float32)),
    )(degp, x_p, Wa)


def _mid_body(a_ref, y0_ref, dinv_ref, l0w_ref, l0b_ref, Wb_ref, y1_ref):
    dinv = dinv_ref[...][:, None]
    t = (a_ref[0] + a_ref[1] + y0_ref[...].astype(jnp.float32)) * dinv
    h = jnp.maximum(t, 0.0)
    h1 = lax.dot_general(h, l0w_ref[...], (((1,), (1,)), ((), ())),
                         precision=_HI) + l0b_ref[...][None, :]
    y1 = lax.dot_general(h1, Wb_ref[...], (((1,), (0,)), ((), ())),
                         precision=_HI) * dinv
    y1_ref[...] = y1.astype(jnp.bfloat16)


def _mid_call(acc, y0, dinv, l0w, l0b, Wb):
    return pl.pallas_call(
        _mid_body,
        grid=(_G,),
        in_specs=[
            pl.BlockSpec((NC, _BLK, D), lambda i: (0, i, 0)),
            pl.BlockSpec((_BLK, D), lambda i: (i, 0)),
            pl.BlockSpec((_BLK,), lambda i: (i,)),
            pl.BlockSpec((D, D), lambda i: (0, 0)),
            pl.BlockSpec((D,), lambda i: (0,)),
            pl.BlockSpec((D, D), lambda i: (0, 0)),
        ],
        out_specs=pl.BlockSpec((_BLK, D), lambda i: (i, 0)),
        out_shape=jax.ShapeDtypeStruct((NP, D), jnp.bfloat16),
    )(acc, y0, dinv, l0w, l0b, Wb)


def _fin_body(a_ref, y1_ref, dinv_ref, l1w_ref, l1b_ref, o_ref):
    dinv = dinv_ref[...][:, None]
    t = (a_ref[0] + a_ref[1] + y1_ref[...].astype(jnp.float32)) * dinv
    o = lax.dot_general(t, l1w_ref[...], (((1,), (1,)), ((), ())),
                        precision=_HI) + l1b_ref[...][None, :]
    o_ref[...] = jax.nn.sigmoid(o)


def _fin_call(acc, y1, dinv, l1w_p, l1b_p):
    return pl.pallas_call(
        _fin_body,
        grid=(_G,),
        in_specs=[
            pl.BlockSpec((NC, _BLK, D), lambda i: (0, i, 0)),
            pl.BlockSpec((_BLK, D), lambda i: (i, 0)),
            pl.BlockSpec((_BLK,), lambda i: (i,)),
            pl.BlockSpec((D, D), lambda i: (0, 0)),
            pl.BlockSpec((D,), lambda i: (0,)),
        ],
        out_specs=pl.BlockSpec((_BLK, D), lambda i: (i, 0)),
        out_shape=jax.ShapeDtypeStruct((NP, D), jnp.float32),
    )(acc, y1, dinv, l1w_p, l1b_p)


def kernel(x, edge_index, edge_weight, W0, gru0_w_ih, gru0_w_hh, gru0_b_ih,
           gru0_b_hh, lin0_w, lin0_b, W1, gru1_w_ih, gru1_w_hh, gru1_b_ih,
           gru1_b_hh, lin1_w, lin1_b):
    src = edge_index[0].astype(jnp.int32)
    dst = edge_index[1].astype(jnp.int32)
    pad = EP - E
    src_p = jnp.concatenate([src, jnp.zeros((pad,), jnp.int32)])
    dst_p = jnp.concatenate([dst, jnp.zeros((pad,), jnp.int32)])
    ew_p = jnp.concatenate([edge_weight, jnp.zeros((pad,), jnp.float32)])
    src3 = src_p.reshape(NW, T, CH)
    dst3 = dst_p.reshape(NW, T, CH)
    dst2 = dst_p.reshape(NW, EPW)
    ew2 = ew_p.reshape(NW, EPW)
    x_p = jnp.concatenate([x, jnp.zeros((NP - N, D), jnp.float32)])
    l1w_p = jnp.zeros((D, D), jnp.float32).at[: lin1_w.shape[0]].set(lin1_w)
    l1b_p = jnp.zeros((D,), jnp.float32).at[: lin1_b.shape[0]].set(lin1_b)

    Wa, Wb = _gru_call(W0, gru0_w_ih, gru0_w_hh, gru0_b_ih, gru0_b_hh,
                       W1, gru1_w_ih, gru1_w_hh, gru1_b_ih, gru1_b_hh)
    degp = _deg_sc(dst2, ew2)
    y0, dinv = _prep_call(degp, x_p, Wa)
    acc0 = _edge_sc(y0, src3, dst3, ew2)
    y1 = _mid_call(acc0, y0, dinv, lin0_w, lin0_b, Wb)
    acc1 = _edge_sc(y1, src3, dst3, ew2)
    o = _fin_call(acc1, y1, dinv, l1w_p, l1b_p)
    return o[:N, : lin1_w.shape[0]]


# speedup vs baseline: 1.2761x; 1.0483x over previous
import functools

import jax
import jax.numpy as jnp
from jax import lax
from jax.experimental import pallas as pl
from jax.experimental.pallas import tpu as pltpu
from jax.experimental.pallas import tpu_sc as plsc

N = 10000
E = 320000
D = 128
NP = 10240
HALF = NP // 2
NC = 2
NS = 16
NW = NC * NS
L = 16
CH = 128
T = 80
TH = T // 2
EPW = T * CH
EP = NW * EPW
ROWS_PER_TILE = NP // NS

_mesh = plsc.VectorSubcoreMesh(core_axis_name="c", subcore_axis_name="s")
_HI = lax.Precision.HIGHEST


@functools.partial(
    pl.kernel,
    mesh=_mesh,
    out_type=jax.ShapeDtypeStruct((NW, NP * L), jnp.float32),
    scratch_types=[
        pltpu.VMEM((EPW,), jnp.int32),
        pltpu.VMEM((EPW,), jnp.float32),
        pltpu.VMEM((HALF * L,), jnp.float32),
    ],
    compiler_params=pltpu.CompilerParams(needs_layout_passes=False),
)
def _deg_sc(dst_hbm, ew_hbm, out_hbm, dst_v, ew_v, degw):
    c = lax.axis_index("c")
    s = lax.axis_index("s")
    w = c * NS + s
    pltpu.sync_copy(dst_hbm.at[w], dst_v)
    pltpu.sync_copy(ew_hbm.at[w], ew_v)
    col = lax.iota(jnp.int32, L)
    for half in range(2):
        lo = half * HALF

        def zbody(i, carry):
            for u in range(8):
                degw[pl.ds((i * 8 + u) * L, L)] = jnp.zeros((L,), jnp.float32)
            return carry

        lax.fori_loop(0, HALF // 8, zbody, 0)

        def ebody(g, carry):
            for u in range(4):
                d = dst_v[pl.ds((g * 4 + u) * L, L)]
                wv = ew_v[pl.ds((g * 4 + u) * L, L)]
                idx = (d - lo) * L + col
                m = (d >= lo) & (d < lo + HALF)
                plsc.addupdate_scatter(degw, [idx], wv, mask=m)
            return carry

        lax.fori_loop(0, EPW // L // 4, ebody, 0)
        pltpu.sync_copy(degw, out_hbm.at[w, pl.ds(lo * L, HALF * L)])


@functools.partial(
    pl.kernel,
    mesh=_mesh,
    out_type=jax.ShapeDtypeStruct((NC, NP, D), jnp.float32),
    scratch_types=[
        pltpu.VMEM((TH, CH), jnp.int32),
        pltpu.VMEM((TH, CH), jnp.int32),
        pltpu.VMEM((TH * CH,), jnp.float32),
        pltpu.VMEM((CH, D), jnp.bfloat16),
        pltpu.VMEM((CH, D), jnp.bfloat16),
        pltpu.VMEM((CH, D), jnp.float32),
        pltpu.VMEM_SHARED((NP, D), jnp.float32),
        pltpu.SemaphoreType.DMA,
        pltpu.SemaphoreType.DMA,
    ],
    compiler_params=pltpu.CompilerParams(needs_layout_passes=False,
                                         use_tc_tiling_on_sc=False),
)
def _edge_sc(y_hbm, src_hbm, dst_hbm, ew_hbm, out_hbm,
             src_v, dst_v, ew_v, rows0_v, rows1_v, rowsf_v, acc_sh,
             sem0, sem1):
    c = lax.axis_index("c")
    s = lax.axis_index("s")
    w = c * NS + s

    def zbody(i, carry):
        for f in range(D // L):
            rowsf_v[i, pl.ds(f * L, L)] = jnp.zeros((L,), jnp.float32)
        return carry

    lax.fori_loop(0, CH, zbody, 0)
    for k in range(ROWS_PER_TILE // CH):
        pltpu.sync_copy(rowsf_v, acc_sh.at[pl.ds(s * ROWS_PER_TILE + k * CH, CH)])

    plsc.subcore_barrier()

    col2 = lax.iota(jnp.int32, L) * 2

    def scale_scatter(h, j, rows_v):
        def ebody(e, icarry):
            wv = plsc.load_gather(
                ew_v, [jnp.full((L,), j * CH + e, jnp.int32)])
            erow = jnp.full((L,), e, jnp.int32)
            for f in range(D // 32):
                xb = rows_v[e, pl.ds(f * 32, 32)]
                xi = plsc.bitcast(xb, jnp.int32)
                fe = plsc.bitcast(lax.shift_left(xi, 16), jnp.float32) * wv
                fo = plsc.bitcast(xi & jnp.int32(-65536), jnp.float32) * wv
                plsc.store_scatter(rowsf_v, [erow, col2 + (32 * f)], fe)
                plsc.store_scatter(rowsf_v, [erow, col2 + (32 * f + 1)], fo)
            return icarry

        lax.fori_loop(0, CH, ebody, 0)
        pltpu.sync_copy(rowsf_v, acc_sh.at[dst_v.at[j]], add=True)

    for h in range(2):
        pltpu.sync_copy(src_hbm.at[w, pl.ds(h * TH, TH)], src_v)
        pltpu.sync_copy(dst_hbm.at[w, pl.ds(h * TH, TH)], dst_v)
        pltpu.sync_copy(ew_hbm.at[w, pl.ds(h * TH * CH, TH * CH)], ew_v)
        pltpu.async_copy(y_hbm.at[src_v.at[0]], rows0_v, sem0)

        def pair(p, carry):
            j0 = p * 2
            j1 = j0 + 1
            pltpu.make_async_copy(y_hbm.at[src_v.at[j0]], rows0_v, sem0).wait()
            pltpu.async_copy(y_hbm.at[src_v.at[j1]], rows1_v, sem1)
            scale_scatter(h, j0, rows0_v)
            pltpu.make_async_copy(y_hbm.at[src_v.at[j1]], rows1_v, sem1).wait()
            jn = lax.rem(j0 + 2, TH)
            pltpu.async_copy(y_hbm.at[src_v.at[jn]], rows0_v, sem0)
            scale_scatter(h, j1, rows1_v)
            return carry

        lax.fori_loop(0, TH // 2, pair, 0)
        pltpu.make_async_copy(y_hbm.at[src_v.at[0]], rows0_v, sem0).wait()
    plsc.subcore_barrier()
    pltpu.sync_copy(acc_sh.at[pl.ds(s * ROWS_PER_TILE, ROWS_PER_TILE)],
                    out_hbm.at[c, pl.ds(s * ROWS_PER_TILE, ROWS_PER_TILE)])


def _gru_math(W, wih, whh, bih, bhh):
    gx = lax.dot_general(W, wih, (((1,), (1,)), ((), ())), precision=_HI)
    gx = gx + bih[None, :]
    gh = lax.dot_general(W, whh, (((1,), (1,)), ((), ())), precision=_HI)
    gh = gh + bhh[None, :]
    d = W.shape[1]
    r = jax.nn.sigmoid(gx[:, :d] + gh[:, :d])
    z = jax.nn.sigmoid(gx[:, d:2 * d] + gh[:, d:2 * d])
    n = jnp.tanh(gx[:, 2 * d:] + r * gh[:, 2 * d:])
    return (1.0 - z) * n + z * W


def _gru_body(W0r, wih0, whh0, bih0, bhh0, W1r, wih1, whh1, bih1, bhh1,
              Wa_ref, Wb_ref):
    Wa_ref[...] = _gru_math(W0r[...], wih0[...], whh0[...], bih0[...], bhh0[...])
    Wb_ref[...] = _gru_math(W1r[...], wih1[...], whh1[...], bih1[...], bhh1[...])


def _gru_call(W0, g0wi, g0wh, g0bi, g0bh, W1, g1wi, g1wh, g1bi, g1bh):
    return pl.pallas_call(
        _gru_body,
        out_shape=(jax.ShapeDtypeStruct((D, D), jnp.float32),
                   jax.ShapeDtypeStruct((D, D), jnp.float32)),
    )(W0, g0wi, g0wh, g0bi, g0bh, W1, g1wi, g1wh, g1bi, g1bh)


_BLK = 1024
_G = NP // _BLK


def _prep_body(degp_ref, x_ref, Wa_ref, y0_ref, dinv_ref):
    degp = degp_ref[...].reshape(NW, _BLK, L)
    deg = jnp.sum(degp, axis=(0, 2)) + 1.0
    dinv = lax.rsqrt(deg)
    xw = lax.dot_general(x_ref[...], Wa_ref[...], (((1,), (0,)), ((), ())),
                         precision=_HI)
    y0_ref[...] = (xw * dinv[:, None]).astype(jnp.bfloat16)
    dinv_ref[...] = dinv


def _prep_call(degp, x_p, Wa):
    return pl.pallas_call(
        _prep_body,
        grid=(_G,),
        in_specs=[
            pl.BlockSpec((NW, _BLK * L), lambda i: (0, i)),
            pl.BlockSpec((_BLK, D), lambda i: (i, 0)),
            pl.BlockSpec((D, D), lambda i: (0, 0)),
        ],
        out_specs=[
            pl.BlockSpec((_BLK, D), lambda i: (i, 0)),
            pl.BlockSpec((_BLK,), lambda i: (i,)),
        ],
        out_shape=(jax.ShapeDtypeStruct((NP, D), jnp.bfloat16),
                   jax.ShapeDtypeStruct((NP,), jnp.float32)),
    )(degp, x_p, Wa)


def _mid_body(a_ref, y0_ref, dinv_ref, l0w_ref, l0b_ref, Wb_ref, y1_ref):
    dinv = dinv_ref[...][:, None]
    t = (a_ref[0] + a_ref[1] + y0_ref[...].astype(jnp.float32)) * dinv
    h = jnp.maximum(t, 0.0)
    h1 = lax.dot_general(h, l0w_ref[...], (((1,), (1,)), ((), ())),
                         precision=_HI) + l0b_ref[...][None, :]
    y1 = lax.dot_general(h1, Wb_ref[...], (((1,), (0,)), ((), ())),
                         precision=_HI) * dinv
    y1_ref[...] = y1.astype(jnp.bfloat16)


def _mid_call(acc, y0, dinv, l0w, l0b, Wb):
    return pl.pallas_call(
        _mid_body,
        grid=(_G,),
        in_specs=[
            pl.BlockSpec((NC, _BLK, D), lambda i: (0, i, 0)),
            pl.BlockSpec((_BLK, D), lambda i: (i, 0)),
            pl.BlockSpec((_BLK,), lambda i: (i,)),
            pl.BlockSpec((D, D), lambda i: (0, 0)),
            pl.BlockSpec((D,), lambda i: (0,)),
            pl.BlockSpec((D, D), lambda i: (0, 0)),
        ],
        out_specs=pl.BlockSpec((_BLK, D), lambda i: (i, 0)),
        out_shape=jax.ShapeDtypeStruct((NP, D), jnp.bfloat16),
    )(acc, y0, dinv, l0w, l0b, Wb)


def _fin_body(a_ref, y1_ref, dinv_ref, l1w_ref, l1b_ref, o_ref):
    dinv = dinv_ref[...][:, None]
    t = (a_ref[0] + a_ref[1] + y1_ref[...].astype(jnp.float32)) * dinv
    o = lax.dot_general(t, l1w_ref[...], (((1,), (1,)), ((), ())),
                        precision=_HI) + l1b_ref[...][None, :]
    o_ref[...] = jax.nn.sigmoid(o)


def _fin_call(acc, y1, dinv, l1w_p, l1b_p):
    return pl.pallas_call(
        _fin_body,
        grid=(_G,),
        in_specs=[
            pl.BlockSpec((NC, _BLK, D), lambda i: (0, i, 0)),
            pl.BlockSpec((_BLK, D), lambda i: (i, 0)),
            pl.BlockSpec((_BLK,), lambda i: (i,)),
            pl.BlockSpec((D, D), lambda i: (0, 0)),
            pl.BlockSpec((D,), lambda i: (0,)),
        ],
        out_specs=pl.BlockSpec((_BLK, D), lambda i: (i, 0)),
        out_shape=jax.ShapeDtypeStruct((NP, D), jnp.float32),
    )(acc, y1, dinv, l1w_p, l1b_p)


def kernel(x, edge_index, edge_weight, W0, gru0_w_ih, gru0_w_hh, gru0_b_ih,
           gru0_b_hh, lin0_w, lin0_b, W1, gru1_w_ih, gru1_w_hh, gru1_b_ih,
           gru1_b_hh, lin1_w, lin1_b):
    src = edge_index[0].astype(jnp.int32)
    dst = edge_index[1].astype(jnp.int32)
    pad = EP - E
    src_p = jnp.concatenate([src, jnp.zeros((pad,), jnp.int32)])
    dst_p = jnp.concatenate([dst, jnp.zeros((pad,), jnp.int32)])
    ew_p = jnp.concatenate([edge_weight, jnp.zeros((pad,), jnp.float32)])
    src3 = src_p.reshape(NW, T, CH)
    dst3 = dst_p.reshape(NW, T, CH)
    dst2 = dst_p.reshape(NW, EPW)
    ew2 = ew_p.reshape(NW, EPW)
    x_p = jnp.concatenate([x, jnp.zeros((NP - N, D), jnp.float32)])
    l1w_p = jnp.zeros((D, D), jnp.float32).at[: lin1_w.shape[0]].set(lin1_w)
    l1b_p = jnp.zeros((D,), jnp.float32).at[: lin1_b.shape[0]].set(lin1_b)

    Wa, Wb = _gru_call(W0, gru0_w_ih, gru0_w_hh, gru0_b_ih, gru0_b_hh,
                       W1, gru1_w_ih, gru1_w_hh, gru1_b_ih, gru1_b_hh)
    degp = _deg_sc(dst2, ew2)
    y0, dinv = _prep_call(degp, x_p, Wa)
    acc0 = _edge_sc(y0, src3, dst3, ew2)
    y1 = _mid_call(acc0, y0, dinv, lin0_w, lin0_b, Wb)
    acc1 = _edge_sc(y1, src3, dst3, ew2)
    o = _fin_call(acc1, y1, dinv, l1w_p, l1b_p)
    return o[:N, : lin1_w.shape[0]]


# speedup vs baseline: 1.4262x; 1.1176x over previous
import functools

import jax
import jax.numpy as jnp
from jax import lax
from jax.experimental import pallas as pl
from jax.experimental.pallas import tpu as pltpu
from jax.experimental.pallas import tpu_sc as plsc

N = 10000
E = 320000
D = 128
NP = 10240
HALF = NP // 2
NC = 2
NS = 16
NW = NC * NS
L = 16
CH = 128
T = 80
TH = T // 2
EPW = T * CH
EP = NW * EPW
ROWS_PER_TILE = NP // NS

_mesh = plsc.VectorSubcoreMesh(core_axis_name="c", subcore_axis_name="s")
_HI = lax.Precision.HIGHEST


@functools.partial(
    pl.kernel,
    mesh=_mesh,
    out_type=jax.ShapeDtypeStruct((NW, NP * L), jnp.float32),
    scratch_types=[
        pltpu.VMEM((EPW,), jnp.int32),
        pltpu.VMEM((EPW,), jnp.float32),
        pltpu.VMEM((HALF * L,), jnp.float32),
    ],
    compiler_params=pltpu.CompilerParams(needs_layout_passes=False),
)
def _deg_sc(dst_hbm, ew_hbm, out_hbm, dst_v, ew_v, degw):
    c = lax.axis_index("c")
    s = lax.axis_index("s")
    w = c * NS + s
    pltpu.sync_copy(dst_hbm.at[w], dst_v)
    pltpu.sync_copy(ew_hbm.at[w], ew_v)
    col = lax.iota(jnp.int32, L)
    for half in range(2):
        lo = half * HALF

        def zbody(i, carry):
            for u in range(8):
                degw[pl.ds((i * 8 + u) * L, L)] = jnp.zeros((L,), jnp.float32)
            return carry

        lax.fori_loop(0, HALF // 8, zbody, 0)

        def ebody(g, carry):
            for u in range(4):
                d = dst_v[pl.ds((g * 4 + u) * L, L)]
                wv = ew_v[pl.ds((g * 4 + u) * L, L)]
                idx = (d - lo) * L + col
                m = (d >= lo) & (d < lo + HALF)
                plsc.addupdate_scatter(degw, [idx], wv, mask=m)
            return carry

        lax.fori_loop(0, EPW // L // 4, ebody, 0)
        pltpu.sync_copy(degw, out_hbm.at[w, pl.ds(lo * L, HALF * L)])


@functools.partial(
    pl.kernel,
    mesh=_mesh,
    out_type=jax.ShapeDtypeStruct((NC, NP, D), jnp.float32),
    scratch_types=[
        pltpu.VMEM((TH, CH), jnp.int32),
        pltpu.VMEM((TH, CH), jnp.int32),
        pltpu.VMEM((TH * CH,), jnp.float32),
        pltpu.VMEM((CH, D), jnp.bfloat16),
        pltpu.VMEM((CH, D), jnp.bfloat16),
        pltpu.VMEM((CH, D), jnp.float32),
        pltpu.VMEM_SHARED((NP, D), jnp.float32),
        pltpu.SemaphoreType.DMA,
        pltpu.SemaphoreType.DMA,
    ],
    compiler_params=pltpu.CompilerParams(needs_layout_passes=False,
                                         use_tc_tiling_on_sc=False),
)
def _edge_sc(y_hbm, src_hbm, dst_hbm, ew_hbm, out_hbm,
             src_v, dst_v, ew_v, rows0_v, rows1_v, rowsf_v, acc_sh,
             sem0, sem1):
    c = lax.axis_index("c")
    s = lax.axis_index("s")
    w = c * NS + s

    def zbody(i, carry):
        for f in range(D // L):
            rowsf_v[i, pl.ds(f * L, L)] = jnp.zeros((L,), jnp.float32)
        return carry

    lax.fori_loop(0, CH, zbody, 0)
    for k in range(ROWS_PER_TILE // CH):
        pltpu.sync_copy(rowsf_v, acc_sh.at[pl.ds(s * ROWS_PER_TILE + k * CH, CH)])

    plsc.subcore_barrier()

    col2 = lax.iota(jnp.int32, L) * 2

    def scale_scatter(h, j, rows_v):
        @plsc.parallel_loop(0, CH, 1, unroll=4)
        def ebody(e):
            wv = plsc.load_gather(
                ew_v, [jnp.full((L,), j * CH + e, jnp.int32)])
            erow = jnp.full((L,), e, jnp.int32)
            for f in range(D // 32):
                xb = rows_v[e, pl.ds(f * 32, 32)]
                xi = plsc.bitcast(xb, jnp.int32)
                fe = plsc.bitcast(lax.shift_left(xi, 16), jnp.float32) * wv
                fo = plsc.bitcast(xi & jnp.int32(-65536), jnp.float32) * wv
                plsc.store_scatter(rowsf_v, [erow, col2 + (32 * f)], fe)
                plsc.store_scatter(rowsf_v, [erow, col2 + (32 * f + 1)], fo)

        pltpu.sync_copy(rowsf_v, acc_sh.at[dst_v.at[j]], add=True)

    for h in range(2):
        pltpu.sync_copy(src_hbm.at[w, pl.ds(h * TH, TH)], src_v)
        pltpu.sync_copy(dst_hbm.at[w, pl.ds(h * TH, TH)], dst_v)
        pltpu.sync_copy(ew_hbm.at[w, pl.ds(h * TH * CH, TH * CH)], ew_v)
        pltpu.async_copy(y_hbm.at[src_v.at[0]], rows0_v, sem0)

        def pair(p, carry):
            j0 = p * 2
            j1 = j0 + 1
            pltpu.make_async_copy(y_hbm.at[src_v.at[j0]], rows0_v, sem0).wait()
            pltpu.async_copy(y_hbm.at[src_v.at[j1]], rows1_v, sem1)
            scale_scatter(h, j0, rows0_v)
            pltpu.make_async_copy(y_hbm.at[src_v.at[j1]], rows1_v, sem1).wait()
            jn = lax.rem(j0 + 2, TH)
            pltpu.async_copy(y_hbm.at[src_v.at[jn]], rows0_v, sem0)
            scale_scatter(h, j1, rows1_v)
            return carry

        lax.fori_loop(0, TH // 2, pair, 0)
        pltpu.make_async_copy(y_hbm.at[src_v.at[0]], rows0_v, sem0).wait()
    plsc.subcore_barrier()
    pltpu.sync_copy(acc_sh.at[pl.ds(s * ROWS_PER_TILE, ROWS_PER_TILE)],
                    out_hbm.at[c, pl.ds(s * ROWS_PER_TILE, ROWS_PER_TILE)])


def _gru_math(W, wih, whh, bih, bhh):
    gx = lax.dot_general(W, wih, (((1,), (1,)), ((), ())), precision=_HI)
    gx = gx + bih[None, :]
    gh = lax.dot_general(W, whh, (((1,), (1,)), ((), ())), precision=_HI)
    gh = gh + bhh[None, :]
    d = W.shape[1]
    r = jax.nn.sigmoid(gx[:, :d] + gh[:, :d])
    z = jax.nn.sigmoid(gx[:, d:2 * d] + gh[:, d:2 * d])
    n = jnp.tanh(gx[:, 2 * d:] + r * gh[:, 2 * d:])
    return (1.0 - z) * n + z * W


def _gru_body(W0r, wih0, whh0, bih0, bhh0, W1r, wih1, whh1, bih1, bhh1,
              Wa_ref, Wb_ref):
    Wa_ref[...] = _gru_math(W0r[...], wih0[...], whh0[...], bih0[...], bhh0[...])
    Wb_ref[...] = _gru_math(W1r[...], wih1[...], whh1[...], bih1[...], bhh1[...])


def _gru_call(W0, g0wi, g0wh, g0bi, g0bh, W1, g1wi, g1wh, g1bi, g1bh):
    return pl.pallas_call(
        _gru_body,
        out_shape=(jax.ShapeDtypeStruct((D, D), jnp.float32),
                   jax.ShapeDtypeStruct((D, D), jnp.float32)),
    )(W0, g0wi, g0wh, g0bi, g0bh, W1, g1wi, g1wh, g1bi, g1bh)


_BLK = 1024
_G = NP // _BLK


def _prep_body(degp_ref, x_ref, Wa_ref, y0_ref, dinv_ref):
    degp = degp_ref[...].reshape(NW, _BLK, L)
    deg = jnp.sum(degp, axis=(0, 2)) + 1.0
    dinv = lax.rsqrt(deg)
    xw = lax.dot_general(x_ref[...], Wa_ref[...], (((1,), (0,)), ((), ())),
                         precision=_HI)
    y0_ref[...] = (xw * dinv[:, None]).astype(jnp.bfloat16)
    dinv_ref[...] = dinv


def _prep_call(degp, x_p, Wa):
    return pl.pallas_call(
        _prep_body,
        grid=(_G,),
        in_specs=[
            pl.BlockSpec((NW, _BLK * L), lambda i: (0, i)),
            pl.BlockSpec((_BLK, D), lambda i: (i, 0)),
            pl.BlockSpec((D, D), lambda i: (0, 0)),
        ],
        out_specs=[
            pl.BlockSpec((_BLK, D), lambda i: (i, 0)),
            pl.BlockSpec((_BLK,), lambda i: (i,)),
        ],
        out_shape=(jax.ShapeDtypeStruct((NP, D), jnp.bfloat16),
                   jax.ShapeDtypeStruct((NP,), jnp.float32)),
    )(degp, x_p, Wa)


def _mid_body(a_ref, y0_ref, dinv_ref, l0w_ref, l0b_ref, Wb_ref, y1_ref):
    dinv = dinv_ref[...][:, None]
    t = (a_ref[0] + a_ref[1] + y0_ref[...].astype(jnp.float32)) * dinv
    h = jnp.maximum(t, 0.0)
    h1 = lax.dot_general(h, l0w_ref[...], (((1,), (1,)), ((), ())),
                         precision=_HI) + l0b_ref[...][None, :]
    y1 = lax.dot_general(h1, Wb_ref[...], (((1,), (0,)), ((), ())),
                         precision=_HI) * dinv
    y1_ref[...] = y1.astype(jnp.bfloat16)


def _mid_call(acc, y0, dinv, l0w, l0b, Wb):
    return pl.pallas_call(
        _mid_body,
        grid=(_G,),
        in_specs=[
            pl.BlockSpec((NC, _BLK, D), lambda i: (0, i, 0)),
            pl.BlockSpec((_BLK, D), lambda i: (i, 0)),
            pl.BlockSpec((_BLK,), lambda i: (i,)),
            pl.BlockSpec((D, D), lambda i: (0, 0)),
            pl.BlockSpec((D,), lambda i: (0,)),
            pl.BlockSpec((D, D), lambda i: (0, 0)),
        ],
        out_specs=pl.BlockSpec((_BLK, D), lambda i: (i, 0)),
        out_shape=jax.ShapeDtypeStruct((NP, D), jnp.bfloat16),
    )(acc, y0, dinv, l0w, l0b, Wb)


def _fin_body(a_ref, y1_ref, dinv_ref, l1w_ref, l1b_ref, o_ref):
    dinv = dinv_ref[...][:, None]
    t = (a_ref[0] + a_ref[1] + y1_ref[...].astype(jnp.float32)) * dinv
    o = lax.dot_general(t, l1w_ref[...], (((1,), (1,)), ((), ())),
                        precision=_HI) + l1b_ref[...][None, :]
    o_ref[...] = jax.nn.sigmoid(o)


def _fin_call(acc, y1, dinv, l1w_p, l1b_p):
    return pl.pallas_call(
        _fin_body,
        grid=(_G,),
        in_specs=[
            pl.BlockSpec((NC, _BLK, D), lambda i: (0, i, 0)),
            pl.BlockSpec((_BLK, D), lambda i: (i, 0)),
            pl.BlockSpec((_BLK,), lambda i: (i,)),
            pl.BlockSpec((D, D), lambda i: (0, 0)),
            pl.BlockSpec((D,), lambda i: (0,)),
        ],
        out_specs=pl.BlockSpec((_BLK, D), lambda i: (i, 0)),
        out_shape=jax.ShapeDtypeStruct((NP, D), jnp.float32),
    )(acc, y1, dinv, l1w_p, l1b_p)


def kernel(x, edge_index, edge_weight, W0, gru0_w_ih, gru0_w_hh, gru0_b_ih,
           gru0_b_hh, lin0_w, lin0_b, W1, gru1_w_ih, gru1_w_hh, gru1_b_ih,
           gru1_b_hh, lin1_w, lin1_b):
    src = edge_index[0].astype(jnp.int32)
    dst = edge_index[1].astype(jnp.int32)
    pad = EP - E
    src_p = jnp.concatenate([src, jnp.zeros((pad,), jnp.int32)])
    dst_p = jnp.concatenate([dst, jnp.zeros((pad,), jnp.int32)])
    ew_p = jnp.concatenate([edge_weight, jnp.zeros((pad,), jnp.float32)])
    src3 = src_p.reshape(NW, T, CH)
    dst3 = dst_p.reshape(NW, T, CH)
    dst2 = dst_p.reshape(NW, EPW)
    ew2 = ew_p.reshape(NW, EPW)
    x_p = jnp.concatenate([x, jnp.zeros((NP - N, D), jnp.float32)])
    l1w_p = jnp.zeros((D, D), jnp.float32).at[: lin1_w.shape[0]].set(lin1_w)
    l1b_p = jnp.zeros((D,), jnp.float32).at[: lin1_b.shape[0]].set(lin1_b)

    Wa, Wb = _gru_call(W0, gru0_w_ih, gru0_w_hh, gru0_b_ih, gru0_b_hh,
                       W1, gru1_w_ih, gru1_w_hh, gru1_b_ih, gru1_b_hh)
    degp = _deg_sc(dst2, ew2)
    y0, dinv = _prep_call(degp, x_p, Wa)
    acc0 = _edge_sc(y0, src3, dst3, ew2)
    y1 = _mid_call(acc0, y0, dinv, lin0_w, lin0_b, Wb)
    acc1 = _edge_sc(y1, src3, dst3, ew2)
    o = _fin_call(acc1, y1, dinv, l1w_p, l1b_p)
    return o[:N, : lin1_w.shape[0]]


# speedup vs baseline: 1.7028x; 1.1939x over previous
import functools

import jax
import jax.numpy as jnp
from jax import lax
from jax.experimental import pallas as pl
from jax.experimental.pallas import tpu as pltpu
from jax.experimental.pallas import tpu_sc as plsc

N = 10000
E = 320000
D = 128
NP = 10240
HALF = NP // 2
NC = 2
NS = 16
NW = NC * NS
L = 16
CH = 128
NCH = 2560
T = NCH // NW
TH = 40
FAST_C = 1
TF = 3 * TH
TS = 1 * TH
EPW = T * CH
EP = NW * EPW
ROWS_PER_TILE = NP // NS

_mesh = plsc.VectorSubcoreMesh(core_axis_name="c", subcore_axis_name="s")
_HI = lax.Precision.HIGHEST


@functools.partial(
    pl.kernel,
    mesh=_mesh,
    out_type=jax.ShapeDtypeStruct((NW, NP * L), jnp.float32),
    scratch_types=[
        pltpu.VMEM((EPW,), jnp.int32),
        pltpu.VMEM((EPW,), jnp.float32),
        pltpu.VMEM((HALF * L,), jnp.float32),
    ],
    compiler_params=pltpu.CompilerParams(needs_layout_passes=False),
)
def _deg_sc(dst_hbm, ew_hbm, out_hbm, dst_v, ew_v, degw):
    c = lax.axis_index("c")
    s = lax.axis_index("s")
    w = c * NS + s
    pltpu.sync_copy(dst_hbm.at[w], dst_v)
    pltpu.sync_copy(ew_hbm.at[w], ew_v)
    col = lax.iota(jnp.int32, L)
    for half in range(2):
        lo = half * HALF

        def zbody(i, carry):
            for u in range(8):
                degw[pl.ds((i * 8 + u) * L, L)] = jnp.zeros((L,), jnp.float32)
            return carry

        lax.fori_loop(0, HALF // 8, zbody, 0)

        def ebody(g, carry):
            for u in range(4):
                d = dst_v[pl.ds((g * 4 + u) * L, L)]
                wv = ew_v[pl.ds((g * 4 + u) * L, L)]
                idx = (d - lo) * L + col
                m = (d >= lo) & (d < lo + HALF)
                plsc.addupdate_scatter(degw, [idx], wv, mask=m)
            return carry

        lax.fori_loop(0, EPW // L // 4, ebody, 0)
        pltpu.sync_copy(degw, out_hbm.at[w, pl.ds(lo * L, HALF * L)])


@functools.partial(
    pl.kernel,
    mesh=_mesh,
    out_type=jax.ShapeDtypeStruct((NC, NP, D), jnp.float32),
    scratch_types=[
        pltpu.VMEM((TH, CH), jnp.int32),
        pltpu.VMEM((TH, CH), jnp.int32),
        pltpu.VMEM((TH * CH,), jnp.float32),
        pltpu.VMEM((CH, D), jnp.bfloat16),
        pltpu.VMEM((CH, D), jnp.bfloat16),
        pltpu.VMEM((CH, D), jnp.float32),
        pltpu.VMEM_SHARED((NP, D), jnp.float32),
        pltpu.SemaphoreType.DMA,
        pltpu.SemaphoreType.DMA,
    ],
    compiler_params=pltpu.CompilerParams(needs_layout_passes=False,
                                         use_tc_tiling_on_sc=False),
)
def _edge_sc(y_hbm, src_hbm, dst_hbm, ew_hbm, out_hbm,
             src_v, dst_v, ew_v, rows0_v, rows1_v, rowsf_v, acc_sh,
             sem0, sem1):
    c = lax.axis_index("c")
    s = lax.axis_index("s")

    def zbody(i, carry):
        for f in range(D // L):
            rowsf_v[i, pl.ds(f * L, L)] = jnp.zeros((L,), jnp.float32)
        return carry

    lax.fori_loop(0, CH, zbody, 0)
    for k in range(ROWS_PER_TILE // CH):
        pltpu.sync_copy(rowsf_v, acc_sh.at[pl.ds(s * ROWS_PER_TILE + k * CH, CH)])

    plsc.subcore_barrier()

    col2 = lax.iota(jnp.int32, L) * 2

    def scale_scatter(j, rows_v):
        @plsc.parallel_loop(0, CH, 1, unroll=4)
        def ebody(e):
            wv = plsc.load_gather(
                ew_v, [jnp.full((L,), j * CH + e, jnp.int32)])
            erow = jnp.full((L,), e, jnp.int32)
            for f in range(D // 32):
                xb = rows_v[e, pl.ds(f * 32, 32)]
                xi = plsc.bitcast(xb, jnp.int32)
                fe = plsc.bitcast(lax.shift_left(xi, 16), jnp.float32) * wv
                fo = plsc.bitcast(xi & jnp.int32(-65536), jnp.float32) * wv
                plsc.store_scatter(rowsf_v, [erow, col2 + (32 * f)], fe)
                plsc.store_scatter(rowsf_v, [erow, col2 + (32 * f + 1)], fo)

        pltpu.sync_copy(rowsf_v, acc_sh.at[dst_v.at[j]], add=True)

    base = jnp.where(c == FAST_C, s * TF, 16 * TF + s * TS)

    def run_stage(row_lo):
        pltpu.sync_copy(src_hbm.at[pl.ds(row_lo, TH)], src_v)
        pltpu.sync_copy(dst_hbm.at[pl.ds(row_lo, TH)], dst_v)
        pltpu.sync_copy(ew_hbm.at[pl.ds(row_lo * CH, TH * CH)], ew_v)
        pltpu.async_copy(y_hbm.at[src_v.at[0]], rows0_v, sem0)

        def pair(p, carry):
            j0 = p * 2
            j1 = j0 + 1
            pltpu.make_async_copy(y_hbm.at[src_v.at[j0]], rows0_v, sem0).wait()
            pltpu.async_copy(y_hbm.at[src_v.at[j1]], rows1_v, sem1)
            scale_scatter(j0, rows0_v)
            pltpu.make_async_copy(y_hbm.at[src_v.at[j1]], rows1_v, sem1).wait()
            jn = lax.rem(j0 + 2, TH)
            pltpu.async_copy(y_hbm.at[src_v.at[jn]], rows0_v, sem0)
            scale_scatter(j1, rows1_v)
            return carry

        lax.fori_loop(0, TH // 2, pair, 0)
        pltpu.make_async_copy(y_hbm.at[src_v.at[0]], rows0_v, sem0).wait()

    run_stage(base)
    for stage in range(1, TF // TH):
        @pl.when(c == FAST_C)
        def _():
            run_stage(base + stage * TH)

    plsc.subcore_barrier()
    pltpu.sync_copy(acc_sh.at[pl.ds(s * ROWS_PER_TILE, ROWS_PER_TILE)],
                    out_hbm.at[c, pl.ds(s * ROWS_PER_TILE, ROWS_PER_TILE)])


def _gru_math(W, wih, whh, bih, bhh):
    gx = lax.dot_general(W, wih, (((1,), (1,)), ((), ())), precision=_HI)
    gx = gx + bih[None, :]
    gh = lax.dot_general(W, whh, (((1,), (1,)), ((), ())), precision=_HI)
    gh = gh + bhh[None, :]
    d = W.shape[1]
    r = jax.nn.sigmoid(gx[:, :d] + gh[:, :d])
    z = jax.nn.sigmoid(gx[:, d:2 * d] + gh[:, d:2 * d])
    n = jnp.tanh(gx[:, 2 * d:] + r * gh[:, 2 * d:])
    return (1.0 - z) * n + z * W


def _gru_body(W0r, wih0, whh0, bih0, bhh0, W1r, wih1, whh1, bih1, bhh1,
              Wa_ref, Wb_ref):
    Wa_ref[...] = _gru_math(W0r[...], wih0[...], whh0[...], bih0[...], bhh0[...])
    Wb_ref[...] = _gru_math(W1r[...], wih1[...], whh1[...], bih1[...], bhh1[...])


def _gru_call(W0, g0wi, g0wh, g0bi, g0bh, W1, g1wi, g1wh, g1bi, g1bh):
    return pl.pallas_call(
        _gru_body,
        out_shape=(jax.ShapeDtypeStruct((D, D), jnp.float32),
                   jax.ShapeDtypeStruct((D, D), jnp.float32)),
    )(W0, g0wi, g0wh, g0bi, g0bh, W1, g1wi, g1wh, g1bi, g1bh)


_BLK = 1024
_G = NP // _BLK


def _prep_body(degp_ref, x_ref, Wa_ref, y0_ref, dinv_ref):
    degp = degp_ref[...].reshape(NW, _BLK, L)
    deg = jnp.sum(degp, axis=(0, 2)) + 1.0
    dinv = lax.rsqrt(deg)
    xw = lax.dot_general(x_ref[...], Wa_ref[...], (((1,), (0,)), ((), ())),
                         precision=_HI)
    y0_ref[...] = (xw * dinv[:, None]).astype(jnp.bfloat16)
    dinv_ref[...] = dinv


def _prep_call(degp, x_p, Wa):
    return pl.pallas_call(
        _prep_body,
        grid=(_G,),
        in_specs=[
            pl.BlockSpec((NW, _BLK * L), lambda i: (0, i)),
            pl.BlockSpec((_BLK, D), lambda i: (i, 0)),
            pl.BlockSpec((D, D), lambda i: (0, 0)),
        ],
        out_specs=[
            pl.BlockSpec((_BLK, D), lambda i: (i, 0)),
            pl.BlockSpec((_BLK,), lambda i: (i,)),
        ],
        out_shape=(jax.ShapeDtypeStruct((NP, D), jnp.bfloat16),
                   jax.ShapeDtypeStruct((NP,), jnp.float32)),
    )(degp, x_p, Wa)


def _mid_body(a_ref, y0_ref, dinv_ref, l0w_ref, l0b_ref, Wb_ref, y1_ref):
    dinv = dinv_ref[...][:, None]
    t = (a_ref[0] + a_ref[1] + y0_ref[...].astype(jnp.float32)) * dinv
    h = jnp.maximum(t, 0.0)
    h1 = lax.dot_general(h, l0w_ref[...], (((1,), (1,)), ((), ())),
                         precision=_HI) + l0b_ref[...][None, :]
    y1 = lax.dot_general(h1, Wb_ref[...], (((1,), (0,)), ((), ())),
                         precision=_HI) * dinv
    y1_ref[...] = y1.astype(jnp.bfloat16)


def _mid_call(acc, y0, dinv, l0w, l0b, Wb):
    return pl.pallas_call(
        _mid_body,
        grid=(_G,),
        in_specs=[
            pl.BlockSpec((NC, _BLK, D), lambda i: (0, i, 0)),
            pl.BlockSpec((_BLK, D), lambda i: (i, 0)),
            pl.BlockSpec((_BLK,), lambda i: (i,)),
            pl.BlockSpec((D, D), lambda i: (0, 0)),
            pl.BlockSpec((D,), lambda i: (0,)),
            pl.BlockSpec((D, D), lambda i: (0, 0)),
        ],
        out_specs=pl.BlockSpec((_BLK, D), lambda i: (i, 0)),
        out_shape=jax.ShapeDtypeStruct((NP, D), jnp.bfloat16),
    )(acc, y0, dinv, l0w, l0b, Wb)


def _fin_body(a_ref, y1_ref, dinv_ref, l1w_ref, l1b_ref, o_ref):
    dinv = dinv_ref[...][:, None]
    t = (a_ref[0] + a_ref[1] + y1_ref[...].astype(jnp.float32)) * dinv
    o = lax.dot_general(t, l1w_ref[...], (((1,), (1,)), ((), ())),
                        precision=_HI) + l1b_ref[...][None, :]
    o_ref[...] = jax.nn.sigmoid(o)


def _fin_call(acc, y1, dinv, l1w_p, l1b_p):
    return pl.pallas_call(
        _fin_body,
        grid=(_G,),
        in_specs=[
            pl.BlockSpec((NC, _BLK, D), lambda i: (0, i, 0)),
            pl.BlockSpec((_BLK, D), lambda i: (i, 0)),
            pl.BlockSpec((_BLK,), lambda i: (i,)),
            pl.BlockSpec((D, D), lambda i: (0, 0)),
            pl.BlockSpec((D,), lambda i: (0,)),
        ],
        out_specs=pl.BlockSpec((_BLK, D), lambda i: (i, 0)),
        out_shape=jax.ShapeDtypeStruct((NP, D), jnp.float32),
    )(acc, y1, dinv, l1w_p, l1b_p)


def kernel(x, edge_index, edge_weight, W0, gru0_w_ih, gru0_w_hh, gru0_b_ih,
           gru0_b_hh, lin0_w, lin0_b, W1, gru1_w_ih, gru1_w_hh, gru1_b_ih,
           gru1_b_hh, lin1_w, lin1_b):
    src = edge_index[0].astype(jnp.int32)
    dst = edge_index[1].astype(jnp.int32)
    pad = EP - E
    src_p = jnp.concatenate([src, jnp.zeros((pad,), jnp.int32)])
    dst_p = jnp.concatenate([dst, jnp.zeros((pad,), jnp.int32)])
    ew_p = jnp.concatenate([edge_weight, jnp.zeros((pad,), jnp.float32)])
    src3 = src_p.reshape(NCH, CH)
    dst3 = dst_p.reshape(NCH, CH)
    dst2 = dst_p.reshape(NW, EPW)
    ew2 = ew_p.reshape(NW, EPW)
    x_p = jnp.concatenate([x, jnp.zeros((NP - N, D), jnp.float32)])
    l1w_p = jnp.zeros((D, D), jnp.float32).at[: lin1_w.shape[0]].set(lin1_w)
    l1b_p = jnp.zeros((D,), jnp.float32).at[: lin1_b.shape[0]].set(lin1_b)

    Wa, Wb = _gru_call(W0, gru0_w_ih, gru0_w_hh, gru0_b_ih, gru0_b_hh,
                       W1, gru1_w_ih, gru1_w_hh, gru1_b_ih, gru1_b_hh)
    degp = _deg_sc(dst2, ew2)
    y0, dinv = _prep_call(degp, x_p, Wa)
    acc0 = _edge_sc(y0, src3, dst3, ew_p)
    y1 = _mid_call(acc0, y0, dinv, lin0_w, lin0_b, Wb)
    acc1 = _edge_sc(y1, src3, dst3, ew_p)
    o = _fin_call(acc1, y1, dinv, l1w_p, l1b_p)
    return o[:N, : lin1_w.shape[0]]


# speedup vs baseline: 1.7514x; 1.0286x over previous
import functools

import jax
import jax.numpy as jnp
from jax import lax
from jax.experimental import pallas as pl
from jax.experimental.pallas import tpu as pltpu
from jax.experimental.pallas import tpu_sc as plsc

N = 10000
E = 320000
D = 128
NP = 10240
HALF = NP // 2
NC = 2
NS = 16
NW = NC * NS
L = 16
CH = 128
NCH = 2560
T = NCH // NW
TH = 40
FAST_C = 1
TF = 108
TS = 52
FAST_STAGES = (36, 36, 36)
SLOW_STAGES = (40, 12)
EPW = T * CH
EP = NW * EPW
ROWS_PER_TILE = NP // NS

_mesh = plsc.VectorSubcoreMesh(core_axis_name="c", subcore_axis_name="s")
_HI = lax.Precision.HIGHEST


@functools.partial(
    pl.kernel,
    mesh=_mesh,
    out_type=jax.ShapeDtypeStruct((NW, NP * L), jnp.float32),
    scratch_types=[
        pltpu.VMEM((EPW,), jnp.int32),
        pltpu.VMEM((EPW,), jnp.float32),
        pltpu.VMEM((HALF * L,), jnp.float32),
    ],
    compiler_params=pltpu.CompilerParams(needs_layout_passes=False),
)
def _deg_sc(dst_hbm, ew_hbm, out_hbm, dst_v, ew_v, degw):
    c = lax.axis_index("c")
    s = lax.axis_index("s")
    w = c * NS + s
    pltpu.sync_copy(dst_hbm.at[w], dst_v)
    pltpu.sync_copy(ew_hbm.at[w], ew_v)
    col = lax.iota(jnp.int32, L)
    for half in range(2):
        lo = half * HALF

        def zbody(i, carry):
            for u in range(8):
                degw[pl.ds((i * 8 + u) * L, L)] = jnp.zeros((L,), jnp.float32)
            return carry

        lax.fori_loop(0, HALF // 8, zbody, 0)

        def ebody(g, carry):
            for u in range(4):
                d = dst_v[pl.ds((g * 4 + u) * L, L)]
                wv = ew_v[pl.ds((g * 4 + u) * L, L)]
                idx = (d - lo) * L + col
                m = (d >= lo) & (d < lo + HALF)
                plsc.addupdate_scatter(degw, [idx], wv, mask=m)
            return carry

        lax.fori_loop(0, EPW // L // 4, ebody, 0)
        pltpu.sync_copy(degw, out_hbm.at[w, pl.ds(lo * L, HALF * L)])


@functools.partial(
    pl.kernel,
    mesh=_mesh,
    out_type=jax.ShapeDtypeStruct((NC, NP, D), jnp.float32),
    scratch_types=[
        pltpu.VMEM((TH, CH), jnp.int32),
        pltpu.VMEM((TH, CH), jnp.int32),
        pltpu.VMEM((TH * CH,), jnp.float32),
        pltpu.VMEM((CH, D), jnp.bfloat16),
        pltpu.VMEM((CH, D), jnp.bfloat16),
        pltpu.VMEM((CH, D), jnp.float32),
        pltpu.VMEM_SHARED((NP, D), jnp.float32),
        pltpu.SemaphoreType.DMA,
        pltpu.SemaphoreType.DMA,
    ],
    compiler_params=pltpu.CompilerParams(needs_layout_passes=False,
                                         use_tc_tiling_on_sc=False),
)
def _edge_sc(y_hbm, src_hbm, dst_hbm, ew_hbm, out_hbm,
             src_v, dst_v, ew_v, rows0_v, rows1_v, rowsf_v, acc_sh,
             sem0, sem1):
    c = lax.axis_index("c")
    s = lax.axis_index("s")

    def zbody(i, carry):
        for f in range(D // L):
            rowsf_v[i, pl.ds(f * L, L)] = jnp.zeros((L,), jnp.float32)
        return carry

    lax.fori_loop(0, CH, zbody, 0)
    for k in range(ROWS_PER_TILE // CH):
        pltpu.sync_copy(rowsf_v, acc_sh.at[pl.ds(s * ROWS_PER_TILE + k * CH, CH)])

    plsc.subcore_barrier()

    col2 = lax.iota(jnp.int32, L) * 2

    def scale_scatter(j, rows_v):
        @plsc.parallel_loop(0, CH, 1, unroll=4)
        def ebody(e):
            wv = plsc.load_gather(
                ew_v, [jnp.full((L,), j * CH + e, jnp.int32)])
            erow = jnp.full((L,), e, jnp.int32)
            for f in range(D // 32):
                xb = rows_v[e, pl.ds(f * 32, 32)]
                xi = plsc.bitcast(xb, jnp.int32)
                fe = plsc.bitcast(lax.shift_left(xi, 16), jnp.float32) * wv
                fo = plsc.bitcast(xi & jnp.int32(-65536), jnp.float32) * wv
                plsc.store_scatter(rowsf_v, [erow, col2 + (32 * f)], fe)
                plsc.store_scatter(rowsf_v, [erow, col2 + (32 * f + 1)], fo)

        pltpu.sync_copy(rowsf_v, acc_sh.at[dst_v.at[j]], add=True)

    base = jnp.where(c == FAST_C, s * TF, 16 * TF + s * TS)

    def run_stage(row_lo, n):
        pltpu.sync_copy(src_hbm.at[pl.ds(row_lo, TH)], src_v)
        pltpu.sync_copy(dst_hbm.at[pl.ds(row_lo, TH)], dst_v)
        pltpu.sync_copy(ew_hbm.at[pl.ds(row_lo * CH, TH * CH)], ew_v)
        pltpu.async_copy(y_hbm.at[src_v.at[0]], rows0_v, sem0)

        def pair(p, carry):
            j0 = p * 2
            j1 = j0 + 1
            pltpu.make_async_copy(y_hbm.at[src_v.at[j0]], rows0_v, sem0).wait()
            pltpu.async_copy(y_hbm.at[src_v.at[j1]], rows1_v, sem1)
            scale_scatter(j0, rows0_v)
            pltpu.make_async_copy(y_hbm.at[src_v.at[j1]], rows1_v, sem1).wait()
            jn = lax.rem(j0 + 2, n)
            pltpu.async_copy(y_hbm.at[src_v.at[jn]], rows0_v, sem0)
            scale_scatter(j1, rows1_v)
            return carry

        lax.fori_loop(0, n // 2, pair, 0)
        pltpu.make_async_copy(y_hbm.at[src_v.at[0]], rows0_v, sem0).wait()

    off_f = 0
    off_s = 0
    nst = max(len(FAST_STAGES), len(SLOW_STAGES))
    for i in range(nst):
        nf = FAST_STAGES[i] if i < len(FAST_STAGES) else 0
        ns_ = SLOW_STAGES[i] if i < len(SLOW_STAGES) else 0
        if nf and ns_ and nf == ns_:
            row = jnp.where(c == FAST_C, base + off_f, base + off_s)
            run_stage(row, nf)
        else:
            if nf:
                @pl.when(c == FAST_C)
                def _():
                    run_stage(base + off_f, nf)
            if ns_:
                @pl.when(c != FAST_C)
                def _():
                    run_stage(base + off_s, ns_)
        off_f += nf
        off_s += ns_

    plsc.subcore_barrier()
    pltpu.sync_copy(acc_sh.at[pl.ds(s * ROWS_PER_TILE, ROWS_PER_TILE)],
                    out_hbm.at[c, pl.ds(s * ROWS_PER_TILE, ROWS_PER_TILE)])


def _gru_math(W, wih, whh, bih, bhh):
    gx = lax.dot_general(W, wih, (((1,), (1,)), ((), ())), precision=_HI)
    gx = gx + bih[None, :]
    gh = lax.dot_general(W, whh, (((1,), (1,)), ((), ())), precision=_HI)
    gh = gh + bhh[None, :]
    d = W.shape[1]
    r = jax.nn.sigmoid(gx[:, :d] + gh[:, :d])
    z = jax.nn.sigmoid(gx[:, d:2 * d] + gh[:, d:2 * d])
    n = jnp.tanh(gx[:, 2 * d:] + r * gh[:, 2 * d:])
    return (1.0 - z) * n + z * W


def _gru_body(W0r, wih0, whh0, bih0, bhh0, W1r, wih1, whh1, bih1, bhh1,
              Wa_ref, Wb_ref):
    Wa_ref[...] = _gru_math(W0r[...], wih0[...], whh0[...], bih0[...], bhh0[...])
    Wb_ref[...] = _gru_math(W1r[...], wih1[...], whh1[...], bih1[...], bhh1[...])


def _gru_call(W0, g0wi, g0wh, g0bi, g0bh, W1, g1wi, g1wh, g1bi, g1bh):
    return pl.pallas_call(
        _gru_body,
        out_shape=(jax.ShapeDtypeStruct((D, D), jnp.float32),
                   jax.ShapeDtypeStruct((D, D), jnp.float32)),
    )(W0, g0wi, g0wh, g0bi, g0bh, W1, g1wi, g1wh, g1bi, g1bh)


_BLK = 1024
_G = NP // _BLK


def _prep_body(degp_ref, x_ref, Wa_ref, y0_ref, dinv_ref):
    degp = degp_ref[...].reshape(NW, _BLK, L)
    deg = jnp.sum(degp, axis=(0, 2)) + 1.0
    dinv = lax.rsqrt(deg)
    xw = lax.dot_general(x_ref[...], Wa_ref[...], (((1,), (0,)), ((), ())),
                         precision=_HI)
    y0_ref[...] = (xw * dinv[:, None]).astype(jnp.bfloat16)
    dinv_ref[...] = dinv


def _prep_call(degp, x_p, Wa):
    return pl.pallas_call(
        _prep_body,
        grid=(_G,),
        in_specs=[
            pl.BlockSpec((NW, _BLK * L), lambda i: (0, i)),
            pl.BlockSpec((_BLK, D), lambda i: (i, 0)),
            pl.BlockSpec((D, D), lambda i: (0, 0)),
        ],
        out_specs=[
            pl.BlockSpec((_BLK, D), lambda i: (i, 0)),
            pl.BlockSpec((_BLK,), lambda i: (i,)),
        ],
        out_shape=(jax.ShapeDtypeStruct((NP, D), jnp.bfloat16),
                   jax.ShapeDtypeStruct((NP,), jnp.float32)),
    )(degp, x_p, Wa)


def _mid_body(a_ref, y0_ref, dinv_ref, l0w_ref, l0b_ref, Wb_ref, y1_ref):
    dinv = dinv_ref[...][:, None]
    t = (a_ref[0] + a_ref[1] + y0_ref[...].astype(jnp.float32)) * dinv
    h = jnp.maximum(t, 0.0)
    h1 = lax.dot_general(h, l0w_ref[...], (((1,), (1,)), ((), ())),
                         precision=_HI) + l0b_ref[...][None, :]
    y1 = lax.dot_general(h1, Wb_ref[...], (((1,), (0,)), ((), ())),
                         precision=_HI) * dinv
    y1_ref[...] = y1.astype(jnp.bfloat16)


def _mid_call(acc, y0, dinv, l0w, l0b, Wb):
    return pl.pallas_call(
        _mid_body,
        grid=(_G,),
        in_specs=[
            pl.BlockSpec((NC, _BLK, D), lambda i: (0, i, 0)),
            pl.BlockSpec((_BLK, D), lambda i: (i, 0)),
            pl.BlockSpec((_BLK,), lambda i: (i,)),
            pl.BlockSpec((D, D), lambda i: (0, 0)),
            pl.BlockSpec((D,), lambda i: (0,)),
            pl.BlockSpec((D, D), lambda i: (0, 0)),
        ],
        out_specs=pl.BlockSpec((_BLK, D), lambda i: (i, 0)),
        out_shape=jax.ShapeDtypeStruct((NP, D), jnp.bfloat16),
    )(acc, y0, dinv, l0w, l0b, Wb)


def _fin_body(a_ref, y1_ref, dinv_ref, l1w_ref, l1b_ref, o_ref):
    dinv = dinv_ref[...][:, None]
    t = (a_ref[0] + a_ref[1] + y1_ref[...].astype(jnp.float32)) * dinv
    o = lax.dot_general(t, l1w_ref[...], (((1,), (1,)), ((), ())),
                        precision=_HI) + l1b_ref[...][None, :]
    o_ref[...] = jax.nn.sigmoid(o)


def _fin_call(acc, y1, dinv, l1w_p, l1b_p):
    return pl.pallas_call(
        _fin_body,
        grid=(_G,),
        in_specs=[
            pl.BlockSpec((NC, _BLK, D), lambda i: (0, i, 0)),
            pl.BlockSpec((_BLK, D), lambda i: (i, 0)),
            pl.BlockSpec((_BLK,), lambda i: (i,)),
            pl.BlockSpec((D, D), lambda i: (0, 0)),
            pl.BlockSpec((D,), lambda i: (0,)),
        ],
        out_specs=pl.BlockSpec((_BLK, D), lambda i: (i, 0)),
        out_shape=jax.ShapeDtypeStruct((NP, D), jnp.float32),
    )(acc, y1, dinv, l1w_p, l1b_p)


def kernel(x, edge_index, edge_weight, W0, gru0_w_ih, gru0_w_hh, gru0_b_ih,
           gru0_b_hh, lin0_w, lin0_b, W1, gru1_w_ih, gru1_w_hh, gru1_b_ih,
           gru1_b_hh, lin1_w, lin1_b):
    src = edge_index[0].astype(jnp.int32)
    dst = edge_index[1].astype(jnp.int32)
    pad = EP - E
    src_p = jnp.concatenate([src, jnp.zeros((pad,), jnp.int32)])
    dst_p = jnp.concatenate([dst, jnp.zeros((pad,), jnp.int32)])
    ew_p = jnp.concatenate([edge_weight, jnp.zeros((pad,), jnp.float32)])
    xpad = jnp.zeros((TH * CH,), jnp.int32)
    src3 = jnp.concatenate([src_p, xpad]).reshape(NCH + TH, CH)
    dst3 = jnp.concatenate([dst_p, xpad]).reshape(NCH + TH, CH)
    ew_px = jnp.concatenate([ew_p, jnp.zeros((TH * CH,), jnp.float32)])
    dst2 = dst_p.reshape(NW, EPW)
    ew2 = ew_p.reshape(NW, EPW)
    x_p = jnp.concatenate([x, jnp.zeros((NP - N, D), jnp.float32)])
    l1w_p = jnp.zeros((D, D), jnp.float32).at[: lin1_w.shape[0]].set(lin1_w)
    l1b_p = jnp.zeros((D,), jnp.float32).at[: lin1_b.shape[0]].set(lin1_b)

    Wa, Wb = _gru_call(W0, gru0_w_ih, gru0_w_hh, gru0_b_ih, gru0_b_hh,
                       W1, gru1_w_ih, gru1_w_hh, gru1_b_ih, gru1_b_hh)
    degp = _deg_sc(dst2, ew2)
    y0, dinv = _prep_call(degp, x_p, Wa)
    acc0 = _edge_sc(y0, src3, dst3, ew_px)
    y1 = _mid_call(acc0, y0, dinv, lin0_w, lin0_b, Wb)
    acc1 = _edge_sc(y1, src3, dst3, ew_px)
    o = _fin_call(acc1, y1, dinv, l1w_p, l1b_p)
    return o[:N, : lin1_w.shape[0]]
